# Initial kernel scaffold; baseline (speedup 1.0000x reference)
import jax
import jax.numpy as jnp
from jax.experimental import pallas as pl


def kernel(x, adj_mask, batch_onehot, wm, bm, wr, wc, be, wg1, bg1, wg2, bg2, wf1a, wf1b, bf1, wf2, bf2):
    raise NotImplementedError("write your pallas kernel here")


# speedup vs baseline: 13.2645x; 13.2645x over previous
import functools

import jax
import jax.numpy as jnp
from jax import lax
from jax.experimental import pallas as pl
from jax.experimental.pallas import tpu as pltpu

NEG = -1e30
VMEM_LIMIT = 32 * 1024 * 1024
K_PRESERVE = 3


def _feat_kernel(x_ref, wm_ref, bm_ref, wr_ref, wc_ref, be_ref,
                 h_ref, arow_ref, ct_ref):
    f32 = jnp.float32
    h = jnp.maximum(jnp.dot(x_ref[...], wm_ref[...], preferred_element_type=f32)
                    + bm_ref[...], 0.0)
    h_ref[...] = h
    arow_ref[...] = jnp.dot(h, wr_ref[...], preferred_element_type=f32)
    ct = lax.dot_general(wc_ref[...], h, (((0,), (1,)), ((), ())),
                         preferred_element_type=f32)
    ct_ref[...] = ct + be_ref[...]


def _masker_kernel(a_ref, m_ref, ct_ref, adj_ref):
    f32 = jnp.float32
    m = m_ref[...]
    z = jnp.where(m > 0, a_ref[...] + ct_ref[...], NEG)

    zmax = jnp.max(z, axis=-1, keepdims=True)
    lo = zmax - 1.0
    hi = zmax
    for _ in range(22):
        mid = 0.5 * (lo + hi)
        s = jnp.sum(jnp.maximum(z - mid, 0.0), axis=-1, keepdims=True)
        gt = s > 1.0
        lo = jnp.where(gt, mid, lo)
        hi = jnp.where(gt, hi, mid)
    mid = 0.5 * (lo + hi)
    supp = m * (z > mid).astype(f32)
    ksz = jnp.sum(supp, axis=-1, keepdims=True)
    ssum = jnp.sum(supp * z, axis=-1, keepdims=True)
    tau = (ssum - 1.0) * pl.reciprocal(jnp.maximum(ksz, 1.0), approx=True)
    adj1 = m * jnp.maximum(z - tau, 0.0)

    m1 = jnp.max(adj1, axis=-1, keepdims=True)
    c1 = jnp.sum((adj1 == m1).astype(f32), axis=-1, keepdims=True)
    r2 = jnp.where(adj1 < m1, adj1, NEG)
    m2 = jnp.max(r2, axis=-1, keepdims=True)
    c2 = jnp.sum((adj1 == m2).astype(f32), axis=-1, keepdims=True)
    r3 = jnp.where(adj1 < m2, r2, NEG)
    m3 = jnp.max(r3, axis=-1, keepdims=True)
    kf = float(K_PRESERVE)
    thr = jnp.where(c1 >= kf, m1, jnp.where(c1 + c2 >= kf, m2, m3))
    adj_ref[...] = jnp.where(adj1 >= thr, adj1, 0.0)


def _gcn_layer_kernel(adj_ref, xin_ref, w_ref, b_ref, out_ref):
    f32 = jnp.float32
    A = adj_ref[...]
    NPAD = A.shape[0]
    rid = lax.broadcasted_iota(jnp.int32, (NPAD, NPAD), 0)
    cid = lax.broadcasted_iota(jnp.int32, (NPAD, NPAD), 1)
    eye = (rid == cid).astype(f32)
    a_hat = A + eye * (A == 0.0).astype(f32)

    ones_c = jnp.ones((NPAD, 1), f32)
    dv = lax.dot_general(a_hat, ones_c, (((0,), (0,)), ((), ())),
                         preferred_element_type=f32)
    dr = lax.dot_general(ones_c, a_hat, (((0,), (0,)), ((), ())),
                         preferred_element_type=f32)
    dinv_c = jnp.where(dv > 0, lax.rsqrt(dv), 0.0)
    dinv_r = jnp.where(dr > 0, lax.rsqrt(dr), 0.0)
    An = a_hat * dinv_c * dinv_r

    xw = jnp.dot(xin_ref[...], w_ref[...], preferred_element_type=f32)
    agg = lax.dot_general(An, xw, (((0,), (0,)), ((), ())),
                          preferred_element_type=f32)
    out_ref[...] = jnp.maximum(agg + b_ref[...], 0.0)


def _pool_kernel(x2_ref, B_ref, Bt_ref,
                 wf1a_ref, wf1b_ref, bf1_ref, wf2_ref, bf2_ref, out_ref):
    f32 = jnp.float32
    x2 = x2_ref[...]
    B = B_ref[...]
    counts = jnp.sum(B, axis=1, keepdims=True)
    meanp = jnp.dot(B, x2, preferred_element_type=f32) * pl.reciprocal(
        jnp.maximum(counts, 1.0), approx=True)

    Bt = Bt_ref[...]
    num_graphs = B.shape[0]
    max_rows = []
    for g in range(num_graphs):
        mask_g = Bt[:, g:g + 1]
        xg = jnp.where(mask_g > 0, x2, NEG)
        max_rows.append(jnp.max(xg, axis=0, keepdims=True))
    maxp = jnp.concatenate(max_rows, axis=0)

    z1 = jnp.maximum(jnp.dot(meanp, wf1a_ref[...], preferred_element_type=f32)
                     + jnp.dot(maxp, wf1b_ref[...], preferred_element_type=f32)
                     + bf1_ref[...], 0.0)
    out_ref[...] = jnp.maximum(jnp.dot(z1, wf2_ref[...], preferred_element_type=f32)
                               + bf2_ref[...], 0.0)


@jax.jit
def kernel(x, adj_mask, batch_onehot, wm, bm, wr, wc, be,
           wg1, bg1, wg2, bg2, wf1a, wf1b, bf1, wf2, bf2):
    f32 = jnp.float32
    N, F = x.shape
    G = batch_onehot.shape[0]
    H = wm.shape[1]

    NP = ((N + 127) // 128) * 128
    xp = jnp.zeros((NP, F), f32).at[:N, :].set(x)
    mp = jnp.zeros((NP, NP), f32).at[:N, :N].set(adj_mask)
    Bp = jnp.zeros((G, NP), f32).at[:, :N].set(batch_onehot)
    Bt = Bp.T

    vmem = pl.BlockSpec(memory_space=pltpu.MemorySpace.VMEM)
    par = ("parallel",)

    TRF = NP // 2 if NP >= 256 else NP
    h, arow, ct = pl.pallas_call(
        _feat_kernel,
        out_shape=(jax.ShapeDtypeStruct((NP, H), f32),
                   jax.ShapeDtypeStruct((NP, 1), f32),
                   jax.ShapeDtypeStruct((1, NP), f32)),
        grid=(NP // TRF,),
        in_specs=[
            pl.BlockSpec((TRF, F), lambda i: (i, 0)),
            vmem, vmem, vmem, vmem, vmem,
        ],
        out_specs=(pl.BlockSpec((TRF, H), lambda i: (i, 0)),
                   pl.BlockSpec((TRF, 1), lambda i: (i, 0)),
                   pl.BlockSpec((1, TRF), lambda i: (0, i))),
        compiler_params=pltpu.CompilerParams(
            dimension_semantics=par, vmem_limit_bytes=VMEM_LIMIT),
    )(xp, wm, bm, wr, wc, be)

    TR = 128 if NP >= 256 else NP
    new_adj = pl.pallas_call(
        _masker_kernel,
        out_shape=jax.ShapeDtypeStruct((NP, NP), f32),
        grid=(NP // TR,),
        in_specs=[
            pl.BlockSpec((TR, 1), lambda i: (i, 0)),
            pl.BlockSpec((TR, NP), lambda i: (i, 0)),
            pl.BlockSpec((1, NP), lambda i: (0, 0)),
        ],
        out_specs=pl.BlockSpec((TR, NP), lambda i: (i, 0)),
        compiler_params=pltpu.CompilerParams(
            dimension_semantics=par, vmem_limit_bytes=VMEM_LIMIT),
    )(arow, mp, ct)

    HB = H // 2
    def gcn_layer(xin, w, b):
        return pl.pallas_call(
            _gcn_layer_kernel,
            out_shape=jax.ShapeDtypeStruct((NP, H), f32),
            grid=(H // HB,),
            in_specs=[
                pl.BlockSpec((NP, NP), lambda j: (0, 0)),
                pl.BlockSpec((NP, H), lambda j: (0, 0)),
                pl.BlockSpec((H, HB), lambda j: (0, j)),
                pl.BlockSpec((1, HB), lambda j: (0, j)),
            ],
            out_specs=pl.BlockSpec((NP, HB), lambda j: (0, j)),
            compiler_params=pltpu.CompilerParams(
                dimension_semantics=par, vmem_limit_bytes=VMEM_LIMIT),
        )(new_adj, xin, w, b)

    x1 = gcn_layer(h, wg1, bg1)
    x2 = gcn_layer(x1, wg2, bg2)

    out = pl.pallas_call(
        _pool_kernel,
        out_shape=jax.ShapeDtypeStruct((G, H), f32),
        in_specs=[vmem] * 8,
        out_specs=vmem,
        compiler_params=pltpu.CompilerParams(vmem_limit_bytes=VMEM_LIMIT),
    )(x2, Bp, Bt, wf1a, wf1b, bf1, wf2, bf2)
    return out


# speedup vs baseline: 14.7099x; 1.1090x over previous
import functools

import jax
import jax.numpy as jnp
from jax import lax
from jax.experimental import pallas as pl
from jax.experimental.pallas import tpu as pltpu

NEG = -1e30
VMEM_LIMIT = 32 * 1024 * 1024
K_PRESERVE = 3


def _feat_kernel(x_ref, wm_ref, bm_ref, wr_ref, wc_ref, be_ref,
                 h_ref, arow_ref, ct_ref):
    f32 = jnp.float32
    h = jnp.maximum(jnp.dot(x_ref[...], wm_ref[...], preferred_element_type=f32)
                    + bm_ref[...], 0.0)
    h_ref[...] = h
    arow_ref[...] = jnp.dot(h, wr_ref[...], preferred_element_type=f32)
    ct = lax.dot_general(wc_ref[...], h, (((0,), (1,)), ((), ())),
                         preferred_element_type=f32)
    ct_ref[...] = ct + be_ref[...]


def _masker_kernel(a_ref, m_ref, ct_ref, adj_ref):
    f32 = jnp.float32
    m = m_ref[...]
    z = jnp.where(m > 0, a_ref[...] + ct_ref[...], NEG)

    zmax = jnp.max(z, axis=-1, keepdims=True)
    lo = zmax - 1.0
    hi = zmax
    for _ in range(22):
        mid = 0.5 * (lo + hi)
        s = jnp.sum(jnp.maximum(z - mid, 0.0), axis=-1, keepdims=True)
        gt = s > 1.0
        lo = jnp.where(gt, mid, lo)
        hi = jnp.where(gt, hi, mid)
    mid = 0.5 * (lo + hi)
    supp = m * (z > mid).astype(f32)
    ksz = jnp.sum(supp, axis=-1, keepdims=True)
    ssum = jnp.sum(supp * z, axis=-1, keepdims=True)
    tau = (ssum - 1.0) * pl.reciprocal(jnp.maximum(ksz, 1.0), approx=True)
    adj1 = m * jnp.maximum(z - tau, 0.0)

    m1 = jnp.max(adj1, axis=-1, keepdims=True)
    c1 = jnp.sum((adj1 == m1).astype(f32), axis=-1, keepdims=True)
    r2 = jnp.where(adj1 < m1, adj1, NEG)
    m2 = jnp.max(r2, axis=-1, keepdims=True)
    c2 = jnp.sum((adj1 == m2).astype(f32), axis=-1, keepdims=True)
    r3 = jnp.where(adj1 < m2, r2, NEG)
    m3 = jnp.max(r3, axis=-1, keepdims=True)
    kf = float(K_PRESERVE)
    thr = jnp.where(c1 >= kf, m1, jnp.where(c1 + c2 >= kf, m2, m3))
    adj_ref[...] = jnp.where(adj1 >= thr, adj1, 0.0)


def _gcn_layer_kernel(adj_ref, xin_ref, w_ref, b_ref, out_ref):
    f32 = jnp.float32
    A = adj_ref[...]
    NPAD = A.shape[0]
    rid = lax.broadcasted_iota(jnp.int32, (NPAD, NPAD), 0)
    cid = lax.broadcasted_iota(jnp.int32, (NPAD, NPAD), 1)
    eye = (rid == cid).astype(f32)
    a_hat = A + eye * (A == 0.0).astype(f32)

    ones_c = jnp.ones((NPAD, 1), f32)
    dv = lax.dot_general(a_hat, ones_c, (((0,), (0,)), ((), ())),
                         preferred_element_type=f32)
    dr = lax.dot_general(ones_c, a_hat, (((0,), (0,)), ((), ())),
                         preferred_element_type=f32)
    dinv_c = jnp.where(dv > 0, lax.rsqrt(dv), 0.0)
    dinv_r = jnp.where(dr > 0, lax.rsqrt(dr), 0.0)
    An = a_hat * dinv_c * dinv_r

    xw = jnp.dot(xin_ref[...], w_ref[...], preferred_element_type=f32)
    agg = lax.dot_general(An, xw, (((0,), (0,)), ((), ())),
                          preferred_element_type=f32)
    out_ref[...] = jnp.maximum(agg + b_ref[...], 0.0)


def _gcn_layer_pool_kernel(adj_ref, xin_ref, w_ref, b_ref, B_ref, Bt_ref,
                           mean_ref, max_ref):
    f32 = jnp.float32
    A = adj_ref[...]
    NPAD = A.shape[0]
    rid = lax.broadcasted_iota(jnp.int32, (NPAD, NPAD), 0)
    cid = lax.broadcasted_iota(jnp.int32, (NPAD, NPAD), 1)
    eye = (rid == cid).astype(f32)
    a_hat = A + eye * (A == 0.0).astype(f32)

    ones_c = jnp.ones((NPAD, 1), f32)
    dv = lax.dot_general(a_hat, ones_c, (((0,), (0,)), ((), ())),
                         preferred_element_type=f32)
    dr = lax.dot_general(ones_c, a_hat, (((0,), (0,)), ((), ())),
                         preferred_element_type=f32)
    dinv_c = jnp.where(dv > 0, lax.rsqrt(dv), 0.0)
    dinv_r = jnp.where(dr > 0, lax.rsqrt(dr), 0.0)
    An = a_hat * dinv_c * dinv_r

    xw = jnp.dot(xin_ref[...], w_ref[...], preferred_element_type=f32)
    agg = lax.dot_general(An, xw, (((0,), (0,)), ((), ())),
                          preferred_element_type=f32)
    x2 = jnp.maximum(agg + b_ref[...], 0.0)

    B = B_ref[...]
    counts = jnp.sum(B, axis=1, keepdims=True)
    mean_ref[...] = jnp.dot(B, x2, preferred_element_type=f32) * pl.reciprocal(
        jnp.maximum(counts, 1.0), approx=True)

    Bt = Bt_ref[...]
    num_graphs = B.shape[0]
    max_rows = []
    for g in range(num_graphs):
        mask_g = Bt[:, g:g + 1]
        xg = jnp.where(mask_g > 0, x2, NEG)
        max_rows.append(jnp.max(xg, axis=0, keepdims=True))
    max_ref[...] = jnp.concatenate(max_rows, axis=0)


def _fc_kernel(mean_ref, max_ref, wf1a_ref, wf1b_ref, bf1_ref,
               wf2_ref, bf2_ref, out_ref):
    f32 = jnp.float32
    z1 = jnp.maximum(
        jnp.dot(mean_ref[...], wf1a_ref[...], preferred_element_type=f32)
        + jnp.dot(max_ref[...], wf1b_ref[...], preferred_element_type=f32)
        + bf1_ref[...], 0.0)
    out_ref[...] = jnp.maximum(jnp.dot(z1, wf2_ref[...], preferred_element_type=f32)
                               + bf2_ref[...], 0.0)


@jax.jit
def kernel(x, adj_mask, batch_onehot, wm, bm, wr, wc, be,
           wg1, bg1, wg2, bg2, wf1a, wf1b, bf1, wf2, bf2):
    f32 = jnp.float32
    N, F = x.shape
    G = batch_onehot.shape[0]
    H = wm.shape[1]

    NP = ((N + 127) // 128) * 128
    if NP == N:
        xp, mp, Bp = x, adj_mask, batch_onehot
    else:
        xp = jnp.zeros((NP, F), f32).at[:N, :].set(x)
        mp = jnp.zeros((NP, NP), f32).at[:N, :N].set(adj_mask)
        Bp = jnp.zeros((G, NP), f32).at[:, :N].set(batch_onehot)
    Bt = Bp.T

    vmem = pl.BlockSpec(memory_space=pltpu.MemorySpace.VMEM)
    par = ("parallel",)

    TRF = NP // 2 if NP >= 256 else NP
    h, arow, ct = pl.pallas_call(
        _feat_kernel,
        out_shape=(jax.ShapeDtypeStruct((NP, H), f32),
                   jax.ShapeDtypeStruct((NP, 1), f32),
                   jax.ShapeDtypeStruct((1, NP), f32)),
        grid=(NP // TRF,),
        in_specs=[
            pl.BlockSpec((TRF, F), lambda i: (i, 0)),
            vmem, vmem, vmem, vmem, vmem,
        ],
        out_specs=(pl.BlockSpec((TRF, H), lambda i: (i, 0)),
                   pl.BlockSpec((TRF, 1), lambda i: (i, 0)),
                   pl.BlockSpec((1, TRF), lambda i: (0, i))),
        compiler_params=pltpu.CompilerParams(
            dimension_semantics=par, vmem_limit_bytes=VMEM_LIMIT),
    )(xp, wm, bm, wr, wc, be)

    TR = NP // 2 if NP >= 256 else NP
    new_adj = pl.pallas_call(
        _masker_kernel,
        out_shape=jax.ShapeDtypeStruct((NP, NP), f32),
        grid=(NP // TR,),
        in_specs=[
            pl.BlockSpec((TR, 1), lambda i: (i, 0)),
            pl.BlockSpec((TR, NP), lambda i: (i, 0)),
            pl.BlockSpec((1, NP), lambda i: (0, 0)),
        ],
        out_specs=pl.BlockSpec((TR, NP), lambda i: (i, 0)),
        compiler_params=pltpu.CompilerParams(
            dimension_semantics=par, vmem_limit_bytes=VMEM_LIMIT),
    )(arow, mp, ct)

    HB = H // 2
    def gcn_layer(xin, w, b):
        return pl.pallas_call(
            _gcn_layer_kernel,
            out_shape=jax.ShapeDtypeStruct((NP, H), f32),
            grid=(H // HB,),
            in_specs=[
                pl.BlockSpec((NP, NP), lambda j: (0, 0)),
                pl.BlockSpec((NP, H), lambda j: (0, 0)),
                pl.BlockSpec((H, HB), lambda j: (0, j)),
                pl.BlockSpec((1, HB), lambda j: (0, j)),
            ],
            out_specs=pl.BlockSpec((NP, HB), lambda j: (0, j)),
            compiler_params=pltpu.CompilerParams(
                dimension_semantics=par, vmem_limit_bytes=VMEM_LIMIT),
        )(new_adj, xin, w, b)

    x1 = gcn_layer(h, wg1, bg1)

    meanp, maxp = pl.pallas_call(
        _gcn_layer_pool_kernel,
        out_shape=(jax.ShapeDtypeStruct((G, H), f32),
                   jax.ShapeDtypeStruct((G, H), f32)),
        grid=(H // HB,),
        in_specs=[
            pl.BlockSpec((NP, NP), lambda j: (0, 0)),
            pl.BlockSpec((NP, H), lambda j: (0, 0)),
            pl.BlockSpec((H, HB), lambda j: (0, j)),
            pl.BlockSpec((1, HB), lambda j: (0, j)),
            pl.BlockSpec((G, NP), lambda j: (0, 0)),
            pl.BlockSpec((NP, G), lambda j: (0, 0)),
        ],
        out_specs=(pl.BlockSpec((G, HB), lambda j: (0, j)),
                   pl.BlockSpec((G, HB), lambda j: (0, j))),
        compiler_params=pltpu.CompilerParams(
            dimension_semantics=par, vmem_limit_bytes=VMEM_LIMIT),
    )(new_adj, x1, wg2, bg2, Bp, Bt)

    out = pl.pallas_call(
        _fc_kernel,
        out_shape=jax.ShapeDtypeStruct((G, H), f32),
        in_specs=[vmem] * 7,
        out_specs=vmem,
        compiler_params=pltpu.CompilerParams(vmem_limit_bytes=VMEM_LIMIT),
    )(meanp, maxp, wf1a, wf1b, bf1, wf2, bf2)
    return out
